# baseline (speedup 1.0000x reference)
import functools

import jax
import jax.numpy as jnp
from jax import lax
from jax.experimental import pallas as pl
from jax.experimental.pallas import tpu as pltpu
from jax.experimental.pallas import tpu_sc as plsc

_NC = 2
_NS = 16
_C = 128


def _build_sc_segsum(half, nloc, ept, h, with_deg):
    n_chunks = ept // _C
    assert n_chunks % 4 == 0
    n_quads = n_chunks // 4
    rptl = nloc // _NS
    assert half % 128 == 0
    nacc = half + 128
    mesh = plsc.VectorSubcoreMesh(core_axis_name="c", subcore_axis_name="s")

    out_type = [jax.ShapeDtypeStruct((_NC * half, h), jnp.float32)]
    scratch = []
    for _ in range(4):
        scratch += [
            pltpu.VMEM((_C,), jnp.int32),
            pltpu.VMEM((_C,), jnp.int32),
            pltpu.VMEM((_C,), jnp.int32),
            pltpu.VMEM((_C, h), jnp.float32),
            pltpu.SemaphoreType.DMA,
        ]
    scratch += [
        pltpu.VMEM((32, h), jnp.float32),
        pltpu.VMEM_SHARED((nacc, h), jnp.float32),
    ]
    if with_deg:
        out_type.append(jax.ShapeDtypeStruct((_NC, _NS, nloc), jnp.float32))
        scratch += [
            pltpu.VMEM((nloc,), jnp.float32),
        ]

    @functools.partial(pl.kernel,
                       out_type=tuple(out_type) if with_deg else out_type[0],
                       mesh=mesh, scratch_types=tuple(scratch),
                       compiler_params=pltpu.CompilerParams(
                           needs_layout_passes=False))
    def sc_kernel(y_hbm, src_hbm, dst_hbm, zrow_hbm, zcol_hbm, *rest):
        if with_deg:
            (agg_out, deg_out, *slots, zstage, acc_sh, ldeg) = rest
        else:
            agg_out, *slots, zstage, acc_sh = rest
        slots = [tuple(slots[i * 5:(i + 1) * 5]) for i in range(4)]
        cid = lax.axis_index("c")
        sid = lax.axis_index("s")
        lo = cid * half

        r0 = sid * rptl
        pltpu.sync_copy(zrow_hbm, zstage)
        for kk in range(-(-(nacc // 32) // _NS)):
            sl = sid + kk * _NS

            @pl.when(sl < nacc // 32)
            def _():
                pltpu.sync_copy(zstage, acc_sh.at[pl.ds(sl * 32, 32)])
        if with_deg:
            pltpu.sync_copy(zcol_hbm, ldeg)
        plsc.subcore_barrier()

        e0 = sid * ept

        def load_and_fire(slot, base):
            sidx, didx, dloc, rows, sem = slot
            pltpu.sync_copy(src_hbm.at[pl.ds(base, _C)], sidx)
            pltpu.sync_copy(dst_hbm.at[pl.ds(base, _C)], didx)
            pltpu.async_copy(y_hbm.at[sidx], rows, sem)

        def remap(slot):
            sidx, didx, dloc, rows, sem = slot
            ones = jnp.ones((16,), jnp.float32)
            for j in range(_C // 16):
                d16 = didx[pl.ds(j * 16, 16)] - lo
                ok = (d16 >= 0) & (d16 < half)
                d16 = jnp.where(ok, d16, half)
                dloc[pl.ds(j * 16, 16)] = d16
                if with_deg:
                    plsc.addupdate_scatter(ldeg, [d16], ones)

        def drain_gather(slot):
            sidx, didx, dloc, rows, sem = slot
            pltpu.make_async_copy(y_hbm.at[sidx], rows, sem).wait()

        def scatter(slot):
            sidx, didx, dloc, rows, sem = slot
            pltpu.sync_copy(rows, acc_sh.at[dloc], add=True)

        load_and_fire(slots[0], e0)
        load_and_fire(slots[1], e0 + _C)

        def body(g, carry):
            qb = e0 + g * 4 * _C
            load_and_fire(slots[2], qb + 2 * _C)
            load_and_fire(slots[3], qb + 3 * _C)
            remap(slots[0])
            remap(slots[1])
            drain_gather(slots[0])
            scatter(slots[0])
            drain_gather(slots[1])
            scatter(slots[1])
            load_and_fire(slots[0], qb + 4 * _C)
            load_and_fire(slots[1], qb + 5 * _C)
            remap(slots[2])
            remap(slots[3])
            drain_gather(slots[2])
            scatter(slots[2])
            drain_gather(slots[3])
            scatter(slots[3])
            return carry

        lax.fori_loop(0, n_quads, body, 0)
        drain_gather(slots[0])
        drain_gather(slots[1])

        if with_deg:
            pltpu.sync_copy(ldeg, deg_out.at[cid, sid])

        plsc.subcore_barrier()

        for kk in range(-(-(half // 128) // _NS)):
            sl = sid + kk * _NS

            @pl.when(sl < half // 128)
            def _():
                pltpu.sync_copy(acc_sh.at[pl.ds(sl * 128, 128)],
                                agg_out.at[pl.ds(lo + sl * 128, 128)])

    return sc_kernel


def _dot_t(a, w):
    return lax.dot_general(a, w, (((1,), (1,)), ((), ())),
                           preferred_element_type=jnp.float32)


def _lin_pair_body(x_ref, wl_ref, wr_ref, b_ref, yl_ref, yr_ref):
    x = x_ref[...]
    yl_ref[...] = _dot_t(x, wl_ref[...])
    yr_ref[...] = _dot_t(x, wr_ref[...]) + b_ref[...][None, :]


def _tc_lin_pair(x, wl, wr, b, bn):
    n, d = x.shape
    h = wl.shape[0]
    grid = pl.cdiv(n, bn)
    return pl.pallas_call(
        _lin_pair_body,
        grid=(grid,),
        in_specs=[
            pl.BlockSpec((bn, d), lambda i: (i, 0)),
            pl.BlockSpec((h, d), lambda i: (0, 0)),
            pl.BlockSpec((h, d), lambda i: (0, 0)),
            pl.BlockSpec((h,), lambda i: (0,)),
        ],
        out_specs=[
            pl.BlockSpec((bn, h), lambda i: (i, 0)),
            pl.BlockSpec((bn, h), lambda i: (i, 0)),
        ],
        out_shape=[
            jax.ShapeDtypeStruct((n, h), jnp.float32),
            jax.ShapeDtypeStruct((n, h), jnp.float32),
        ],
    )(x, wl, wr, b)


def _mid_body(agg_ref, deg_ref, r1_ref, wl_ref, wr_ref, b_ref,
              y2_ref, r2_ref, rdeg_ref):
    deg = jnp.sum(deg_ref[0], axis=0)
    rdeg = 1.0 / jnp.maximum(deg, 1.0)
    h = jnp.maximum(agg_ref[...] * rdeg[:, None] + r1_ref[...], 0.0)
    y2_ref[...] = _dot_t(h, wl_ref[...])
    r2_ref[...] = _dot_t(h, wr_ref[...]) + b_ref[...][None, :]
    rdeg_ref[...] = rdeg


def _tc_mid(agg, deg, r1, wl, wr, b, bn):
    n, h = r1.shape
    o = wl.shape[0]
    half = deg.shape[2]
    assert half % bn == 0
    nbh = half // bn
    grid = pl.cdiv(n, bn)
    return pl.pallas_call(
        _mid_body,
        grid=(grid,),
        in_specs=[
            pl.BlockSpec((bn, h), lambda i: (i, 0)),
            pl.BlockSpec((1, _NS, bn), lambda i: (i // nbh, 0, i % nbh)),
            pl.BlockSpec((bn, h), lambda i: (i, 0)),
            pl.BlockSpec((o, h), lambda i: (0, 0)),
            pl.BlockSpec((o, h), lambda i: (0, 0)),
            pl.BlockSpec((o,), lambda i: (0,)),
        ],
        out_specs=[
            pl.BlockSpec((bn, o), lambda i: (i, 0)),
            pl.BlockSpec((bn, o), lambda i: (i, 0)),
            pl.BlockSpec((bn,), lambda i: (i,)),
        ],
        out_shape=[
            jax.ShapeDtypeStruct((n, o), jnp.float32),
            jax.ShapeDtypeStruct((n, o), jnp.float32),
            jax.ShapeDtypeStruct((n,), jnp.float32),
        ],
    )(agg, deg, r1, wl, wr, b)


def _final_body(agg_ref, rdeg_ref, r2_ref, out_ref):
    out_ref[...] = agg_ref[...] * rdeg_ref[...][:, None] + r2_ref[...]


def _tc_final(agg, rdeg, r2, bn):
    n, o = r2.shape
    grid = pl.cdiv(n, bn)
    return pl.pallas_call(
        _final_body,
        grid=(grid,),
        in_specs=[
            pl.BlockSpec((bn, o), lambda i: (i, 0)),
            pl.BlockSpec((bn,), lambda i: (i,)),
            pl.BlockSpec((bn, o), lambda i: (i, 0)),
        ],
        out_specs=pl.BlockSpec((bn, o), lambda i: (i, 0)),
        out_shape=jax.ShapeDtypeStruct((n, o), jnp.float32),
    )(agg, rdeg, r2)


def kernel(x, edge_index, W1l, b1, W1r, W2l, b2, W2r):
    n, d = x.shape
    h = W1l.shape[0]

    e = edge_index.shape[1]
    ept = -(-e // (_NS * 4 * _C)) * 4 * _C
    e_pad = ept * _NS + 2 * _C
    src = edge_index[0]
    dst = edge_index[1]
    if e_pad != e:
        pad = e_pad - e
        src = jnp.concatenate([src, jnp.zeros((pad,), jnp.int32)])
        dst = jnp.concatenate([dst, jnp.full((pad,), n, jnp.int32)])

    half = -(-n // (2 * 128)) * 128
    nloc = -(-(half + 1) // 2048) * 2048

    sc1 = _build_sc_segsum(half, nloc, ept, h, with_deg=True)
    sc2 = _build_sc_segsum(half, nloc, ept, h, with_deg=False)
    zrow = jnp.zeros((32, h), jnp.float32)
    zcol = jnp.zeros((nloc,), jnp.float32)

    bn = 512

    y1, r1 = _tc_lin_pair(x, W1l, W1r, b1, bn)
    agg1, dgp = sc1(y1, src, dst, zrow, zcol)
    agg1 = agg1[:n]
    y2, r2, rdeg = _tc_mid(agg1, dgp[:, :, :half], r1, W2l, W2r, b2, bn)
    agg2 = sc2(y2, src, dst, zrow, zcol)
    agg2 = agg2[:n]
    return _tc_final(agg2, rdeg, r2, bn)


# speedup vs baseline: 1.5167x; 1.5167x over previous
import functools

import jax
import jax.numpy as jnp
from jax import lax
from jax.experimental import pallas as pl
from jax.experimental.pallas import tpu as pltpu
from jax.experimental.pallas import tpu_sc as plsc

_NC = 2
_NS = 16
_C = 128


def _build_sc_segsum(half, nloc, ept, h, with_deg):
    n_chunks = ept // _C
    rptl = nloc // _NS
    assert half % 128 == 0
    nacc = half + 128
    mesh = plsc.VectorSubcoreMesh(core_axis_name="c", subcore_axis_name="s")

    out_type = [jax.ShapeDtypeStruct((_NC * half, h), jnp.float32)]
    scratch = [
        pltpu.VMEM((_C,), jnp.int32),
        pltpu.VMEM((_C,), jnp.int32),
        pltpu.VMEM((_C,), jnp.int32),
        pltpu.VMEM((_C, h), jnp.float32),
        pltpu.SemaphoreType.DMA,
        pltpu.VMEM((32, h), jnp.float32),
        pltpu.VMEM_SHARED((nacc, h), jnp.float32),
    ]
    if with_deg:
        out_type.append(jax.ShapeDtypeStruct((_NC, _NS, nloc), jnp.float32))
        scratch += [
            pltpu.VMEM((nloc,), jnp.float32),
        ]

    @functools.partial(pl.kernel,
                       out_type=tuple(out_type) if with_deg else out_type[0],
                       mesh=mesh, scratch_types=tuple(scratch),
                       compiler_params=pltpu.CompilerParams(
                           needs_layout_passes=False))
    def sc_kernel(y_hbm, src_hbm, dst_hbm, zrow_hbm, zcol_hbm, *rest):
        if with_deg:
            (agg_out, deg_out, sidx, didx, dloc, rows, sem,
             zstage, acc_sh, ldeg) = rest
        else:
            agg_out, sidx, didx, dloc, rows, sem, zstage, acc_sh = rest
        cid = lax.axis_index("c")
        sid = lax.axis_index("s")
        lo = cid * half

        r0 = sid * rptl
        pltpu.sync_copy(zrow_hbm, zstage)
        for kk in range(-(-(nacc // 32) // _NS)):
            sl = sid + kk * _NS

            @pl.when(sl < nacc // 32)
            def _():
                pltpu.sync_copy(zstage, acc_sh.at[pl.ds(sl * 32, 32)])
        if with_deg:
            pltpu.sync_copy(zcol_hbm, ldeg)
        plsc.subcore_barrier()

        e0 = sid * ept
        lane = lax.iota(jnp.int32, 16)

        def body(k, carry):
            base = e0 + k * _C
            pltpu.sync_copy(src_hbm.at[pl.ds(base, _C)], sidx)
            pltpu.sync_copy(dst_hbm.at[pl.ds(base, _C)], didx)
            gather = pltpu.async_copy(y_hbm.at[sidx], rows, sem)
            ones = jnp.ones((16,), jnp.float32)
            for j in range(_C // 16):
                d16 = didx[pl.ds(j * 16, 16)] - lo
                ok = (d16 >= 0) & (d16 < half)
                d16 = jnp.where(ok, d16, half + j * 16 + lane)
                dloc[pl.ds(j * 16, 16)] = d16
                if with_deg:
                    plsc.addupdate_scatter(ldeg, [d16], ones)
            gather.wait()
            pltpu.sync_copy(rows, acc_sh.at[dloc], add=True)
            return carry

        lax.fori_loop(0, n_chunks, body, 0)

        if with_deg:
            pltpu.sync_copy(ldeg, deg_out.at[cid, sid])

        plsc.subcore_barrier()

        for kk in range(-(-(half // 128) // _NS)):
            sl = sid + kk * _NS

            @pl.when(sl < half // 128)
            def _():
                pltpu.sync_copy(acc_sh.at[pl.ds(sl * 128, 128)],
                                agg_out.at[pl.ds(lo + sl * 128, 128)])

    return sc_kernel


def _dot_t(a, w):
    return lax.dot_general(a, w, (((1,), (1,)), ((), ())),
                           preferred_element_type=jnp.float32)


def _lin_pair_body(x_ref, wl_ref, wr_ref, b_ref, yl_ref, yr_ref):
    x = x_ref[...]
    yl_ref[...] = _dot_t(x, wl_ref[...])
    yr_ref[...] = _dot_t(x, wr_ref[...]) + b_ref[...][None, :]


def _tc_lin_pair(x, wl, wr, b, bn):
    n, d = x.shape
    h = wl.shape[0]
    grid = pl.cdiv(n, bn)
    return pl.pallas_call(
        _lin_pair_body,
        grid=(grid,),
        in_specs=[
            pl.BlockSpec((bn, d), lambda i: (i, 0)),
            pl.BlockSpec((h, d), lambda i: (0, 0)),
            pl.BlockSpec((h, d), lambda i: (0, 0)),
            pl.BlockSpec((h,), lambda i: (0,)),
        ],
        out_specs=[
            pl.BlockSpec((bn, h), lambda i: (i, 0)),
            pl.BlockSpec((bn, h), lambda i: (i, 0)),
        ],
        out_shape=[
            jax.ShapeDtypeStruct((n, h), jnp.float32),
            jax.ShapeDtypeStruct((n, h), jnp.float32),
        ],
    )(x, wl, wr, b)


def _mid_body(agg_ref, deg_ref, r1_ref, wl_ref, wr_ref, b_ref,
              y2_ref, r2_ref, rdeg_ref):
    deg = jnp.sum(deg_ref[0], axis=0)
    rdeg = 1.0 / jnp.maximum(deg, 1.0)
    h = jnp.maximum(agg_ref[...] * rdeg[:, None] + r1_ref[...], 0.0)
    y2_ref[...] = _dot_t(h, wl_ref[...])
    r2_ref[...] = _dot_t(h, wr_ref[...]) + b_ref[...][None, :]
    rdeg_ref[...] = rdeg


def _tc_mid(agg, deg, r1, wl, wr, b, bn):
    n, h = r1.shape
    o = wl.shape[0]
    half = deg.shape[2]
    assert half % bn == 0
    nbh = half // bn
    grid = pl.cdiv(n, bn)
    return pl.pallas_call(
        _mid_body,
        grid=(grid,),
        in_specs=[
            pl.BlockSpec((bn, h), lambda i: (i, 0)),
            pl.BlockSpec((1, _NS, bn), lambda i: (i // nbh, 0, i % nbh)),
            pl.BlockSpec((bn, h), lambda i: (i, 0)),
            pl.BlockSpec((o, h), lambda i: (0, 0)),
            pl.BlockSpec((o, h), lambda i: (0, 0)),
            pl.BlockSpec((o,), lambda i: (0,)),
        ],
        out_specs=[
            pl.BlockSpec((bn, o), lambda i: (i, 0)),
            pl.BlockSpec((bn, o), lambda i: (i, 0)),
            pl.BlockSpec((bn,), lambda i: (i,)),
        ],
        out_shape=[
            jax.ShapeDtypeStruct((n, o), jnp.float32),
            jax.ShapeDtypeStruct((n, o), jnp.float32),
            jax.ShapeDtypeStruct((n,), jnp.float32),
        ],
    )(agg, deg, r1, wl, wr, b)


def _final_body(agg_ref, rdeg_ref, r2_ref, out_ref):
    out_ref[...] = agg_ref[...] * rdeg_ref[...][:, None] + r2_ref[...]


def _tc_final(agg, rdeg, r2, bn):
    n, o = r2.shape
    grid = pl.cdiv(n, bn)
    return pl.pallas_call(
        _final_body,
        grid=(grid,),
        in_specs=[
            pl.BlockSpec((bn, o), lambda i: (i, 0)),
            pl.BlockSpec((bn,), lambda i: (i,)),
            pl.BlockSpec((bn, o), lambda i: (i, 0)),
        ],
        out_specs=pl.BlockSpec((bn, o), lambda i: (i, 0)),
        out_shape=jax.ShapeDtypeStruct((n, o), jnp.float32),
    )(agg, rdeg, r2)


def kernel(x, edge_index, W1l, b1, W1r, W2l, b2, W2r):
    n, d = x.shape
    h = W1l.shape[0]

    e = edge_index.shape[1]
    ept = -(-e // (_NS * _C)) * _C
    e_pad = ept * _NS
    src = edge_index[0]
    dst = edge_index[1]
    if e_pad != e:
        pad = e_pad - e
        src = jnp.concatenate([src, jnp.zeros((pad,), jnp.int32)])
        dst = jnp.concatenate([dst, jnp.full((pad,), n, jnp.int32)])

    half = -(-n // (2 * 128)) * 128
    nloc = -(-(half + 1) // 2048) * 2048

    sc1 = _build_sc_segsum(half, nloc, ept, h, with_deg=True)
    sc2 = _build_sc_segsum(half, nloc, ept, h, with_deg=False)
    zrow = jnp.zeros((32, h), jnp.float32)
    zcol = jnp.zeros((nloc,), jnp.float32)

    bn = 512

    y1, r1 = _tc_lin_pair(x, W1l, W1r, b1, bn)
    agg1, dgp = sc1(y1, src, dst, zrow, zcol)
    agg1 = agg1[:n]
    y2, r2, rdeg = _tc_mid(agg1, dgp[:, :, :half], r1, W2l, W2r, b2, bn)
    agg2 = sc2(y2, src, dst, zrow, zcol)
    agg2 = agg2[:n]
    return _tc_final(agg2, rdeg, r2, bn)


# speedup vs baseline: 1.6868x; 1.1122x over previous
import functools

import jax
import jax.numpy as jnp
from jax import lax
from jax.experimental import pallas as pl
from jax.experimental.pallas import tpu as pltpu
from jax.experimental.pallas import tpu_sc as plsc

_NC = 2
_NS = 16
_C = 128


def _build_sc_segsum(half, nloc, ept, h, with_deg):
    n_chunks = ept // _C
    assert n_chunks % 2 == 0
    n_pairs = n_chunks // 2
    rptl = nloc // _NS
    assert half % 128 == 0
    nacc = half + 128
    mesh = plsc.VectorSubcoreMesh(core_axis_name="c", subcore_axis_name="s")

    out_type = [jax.ShapeDtypeStruct((_NC * half, h), jnp.float32)]
    scratch = [
        pltpu.VMEM((2, _C), jnp.int32),
        pltpu.VMEM((_C,), jnp.int32),
        pltpu.VMEM((_C, h), jnp.float32),
        pltpu.SemaphoreType.DMA,
        pltpu.VMEM((2, _C), jnp.int32),
        pltpu.VMEM((_C,), jnp.int32),
        pltpu.VMEM((_C, h), jnp.float32),
        pltpu.SemaphoreType.DMA,
        pltpu.VMEM((32, h), jnp.float32),
        pltpu.VMEM_SHARED((nacc, h), jnp.float32),
    ]
    if with_deg:
        out_type.append(jax.ShapeDtypeStruct((_NC, _NS, nloc), jnp.float32))
        scratch += [
            pltpu.VMEM((nloc,), jnp.float32),
        ]

    @functools.partial(pl.kernel,
                       out_type=tuple(out_type) if with_deg else out_type[0],
                       mesh=mesh, scratch_types=tuple(scratch),
                       compiler_params=pltpu.CompilerParams(
                           needs_layout_passes=False))
    def sc_kernel(y_hbm, edges_hbm, zrow_hbm, zcol_hbm, *rest):
        if with_deg:
            (agg_out, deg_out, ei0, dl0, rw0, se0, ei1, dl1, rw1, se1,
             zstage, acc_sh, ldeg) = rest
        else:
            (agg_out, ei0, dl0, rw0, se0, ei1, dl1, rw1, se1,
             zstage, acc_sh) = rest
        slot0 = (ei0, dl0, rw0, se0)
        slot1 = (ei1, dl1, rw1, se1)
        cid = lax.axis_index("c")
        sid = lax.axis_index("s")
        lo = cid * half

        r0 = sid * rptl
        pltpu.sync_copy(zrow_hbm, zstage)
        for kk in range(-(-(nacc // 32) // _NS)):
            sl = sid + kk * _NS

            @pl.when(sl < nacc // 32)
            def _():
                pltpu.sync_copy(zstage, acc_sh.at[pl.ds(sl * 32, 32)])
        if with_deg:
            pltpu.sync_copy(zcol_hbm, ldeg)
        plsc.subcore_barrier()

        e0 = sid * ept
        lane = lax.iota(jnp.int32, 16)

        def load_fire(slot, base):
            ei, dl, rw, se = slot
            pltpu.sync_copy(edges_hbm.at[:, pl.ds(base, _C)], ei)
            pltpu.async_copy(y_hbm.at[ei.at[0]], rw, se)

        def remap(slot):
            ei, dl, rw, se = slot
            ones = jnp.ones((16,), jnp.float32)
            for j in range(_C // 16):
                d16 = ei[1, pl.ds(j * 16, 16)] - lo
                ok = (d16 >= 0) & (d16 < half)
                d16 = jnp.where(ok, d16, half + j * 16 + lane)
                dl[pl.ds(j * 16, 16)] = d16
                if with_deg:
                    plsc.addupdate_scatter(ldeg, [d16], ones)

        def drain_scatter(slot):
            ei, dl, rw, se = slot
            pltpu.make_async_copy(y_hbm.at[ei.at[0]], rw, se).wait()
            pltpu.sync_copy(rw, acc_sh.at[dl], add=True)

        load_fire(slot0, e0)

        def body(p, carry):
            b = e0 + 2 * p * _C
            load_fire(slot1, b + _C)
            remap(slot0)
            drain_scatter(slot0)
            load_fire(slot0, b + 2 * _C)
            remap(slot1)
            drain_scatter(slot1)
            return carry

        lax.fori_loop(0, n_pairs, body, 0)
        ei0, dl0, rw0, se0 = slot0
        pltpu.make_async_copy(y_hbm.at[ei0.at[0]], rw0, se0).wait()

        if with_deg:
            pltpu.sync_copy(ldeg, deg_out.at[cid, sid])

        plsc.subcore_barrier()

        for kk in range(-(-(half // 128) // _NS)):
            sl = sid + kk * _NS

            @pl.when(sl < half // 128)
            def _():
                pltpu.sync_copy(acc_sh.at[pl.ds(sl * 128, 128)],
                                agg_out.at[pl.ds(lo + sl * 128, 128)])

    return sc_kernel


def _dot_t(a, w):
    return lax.dot_general(a, w, (((1,), (1,)), ((), ())),
                           preferred_element_type=jnp.float32)


def _lin_pair_body(x_ref, wl_ref, wr_ref, b_ref, yl_ref, yr_ref):
    x = x_ref[...]
    yl_ref[...] = _dot_t(x, wl_ref[...])
    yr_ref[...] = _dot_t(x, wr_ref[...]) + b_ref[...][None, :]


def _tc_lin_pair(x, wl, wr, b, bn):
    n, d = x.shape
    h = wl.shape[0]
    grid = pl.cdiv(n, bn)
    return pl.pallas_call(
        _lin_pair_body,
        grid=(grid,),
        in_specs=[
            pl.BlockSpec((bn, d), lambda i: (i, 0)),
            pl.BlockSpec((h, d), lambda i: (0, 0)),
            pl.BlockSpec((h, d), lambda i: (0, 0)),
            pl.BlockSpec((h,), lambda i: (0,)),
        ],
        out_specs=[
            pl.BlockSpec((bn, h), lambda i: (i, 0)),
            pl.BlockSpec((bn, h), lambda i: (i, 0)),
        ],
        out_shape=[
            jax.ShapeDtypeStruct((n, h), jnp.float32),
            jax.ShapeDtypeStruct((n, h), jnp.float32),
        ],
    )(x, wl, wr, b)


def _mid_body(agg_ref, deg_ref, r1_ref, wl_ref, wr_ref, b_ref,
              y2_ref, r2_ref, rdeg_ref):
    deg = jnp.sum(deg_ref[0], axis=0)
    rdeg = 1.0 / jnp.maximum(deg, 1.0)
    h = jnp.maximum(agg_ref[...] * rdeg[:, None] + r1_ref[...], 0.0)
    y2_ref[...] = _dot_t(h, wl_ref[...])
    r2_ref[...] = _dot_t(h, wr_ref[...]) + b_ref[...][None, :]
    rdeg_ref[...] = rdeg


def _tc_mid(agg, deg, r1, wl, wr, b, bn):
    n, h = r1.shape
    o = wl.shape[0]
    half = deg.shape[2]
    assert half % bn == 0
    nbh = half // bn
    grid = pl.cdiv(n, bn)
    return pl.pallas_call(
        _mid_body,
        grid=(grid,),
        in_specs=[
            pl.BlockSpec((bn, h), lambda i: (i, 0)),
            pl.BlockSpec((1, _NS, bn), lambda i: (i // nbh, 0, i % nbh)),
            pl.BlockSpec((bn, h), lambda i: (i, 0)),
            pl.BlockSpec((o, h), lambda i: (0, 0)),
            pl.BlockSpec((o, h), lambda i: (0, 0)),
            pl.BlockSpec((o,), lambda i: (0,)),
        ],
        out_specs=[
            pl.BlockSpec((bn, o), lambda i: (i, 0)),
            pl.BlockSpec((bn, o), lambda i: (i, 0)),
            pl.BlockSpec((bn,), lambda i: (i,)),
        ],
        out_shape=[
            jax.ShapeDtypeStruct((n, o), jnp.float32),
            jax.ShapeDtypeStruct((n, o), jnp.float32),
            jax.ShapeDtypeStruct((n,), jnp.float32),
        ],
    )(agg, deg, r1, wl, wr, b)


def _final_body(agg_ref, rdeg_ref, r2_ref, out_ref):
    out_ref[...] = agg_ref[...] * rdeg_ref[...][:, None] + r2_ref[...]


def _tc_final(agg, rdeg, r2, bn):
    n, o = r2.shape
    grid = pl.cdiv(n, bn)
    return pl.pallas_call(
        _final_body,
        grid=(grid,),
        in_specs=[
            pl.BlockSpec((bn, o), lambda i: (i, 0)),
            pl.BlockSpec((bn,), lambda i: (i,)),
            pl.BlockSpec((bn, o), lambda i: (i, 0)),
        ],
        out_specs=pl.BlockSpec((bn, o), lambda i: (i, 0)),
        out_shape=jax.ShapeDtypeStruct((n, o), jnp.float32),
    )(agg, rdeg, r2)


def kernel(x, edge_index, W1l, b1, W1r, W2l, b2, W2r):
    n, d = x.shape
    h = W1l.shape[0]

    e = edge_index.shape[1]
    ept = -(-e // (_NS * 2 * _C)) * 2 * _C
    e_pad = ept * _NS + _C
    pad = e_pad - e
    edges = jnp.concatenate(
        [edge_index,
         jnp.stack([jnp.zeros((pad,), jnp.int32),
                    jnp.full((pad,), n, jnp.int32)])], axis=1)

    half = -(-n // (2 * 128)) * 128
    nloc = -(-(half + 1) // 2048) * 2048

    sc1 = _build_sc_segsum(half, nloc, ept, h, with_deg=True)
    sc2 = _build_sc_segsum(half, nloc, ept, h, with_deg=False)
    zrow = jnp.zeros((32, h), jnp.float32)
    zcol = jnp.zeros((nloc,), jnp.float32)

    bn = 512

    y1, r1 = _tc_lin_pair(x, W1l, W1r, b1, bn)
    agg1, dgp = sc1(y1, edges, zrow, zcol)
    agg1 = agg1[:n]
    y2, r2, rdeg = _tc_mid(agg1, dgp[:, :, :half], r1, W2l, W2r, b2, bn)
    agg2 = sc2(y2, edges, zrow, zcol)
    agg2 = agg2[:n]
    return _tc_final(agg2, rdeg, r2, bn)
